# Initial kernel scaffold; baseline (speedup 1.0000x reference)
import jax
import jax.numpy as jnp
from jax.experimental import pallas as pl


def kernel(x, edge_index, W1, b1, W2, b2, Wd, bd):
    raise NotImplementedError("write your pallas kernel here")


# speedup vs baseline: 2.3822x; 2.3822x over previous
import functools

import jax
import jax.numpy as jnp
import numpy as np
from jax.experimental import pallas as pl
from jax.experimental.pallas import tpu as pltpu

N = 10000
E = 160000
D_IN = 256
D_H = 512
N_CLS = 40

BN = 1000


def _mm_scale_kernel(x_ref, w_ref, b_ref, dinv_ref, o_ref):
    acc = jnp.dot(x_ref[...], w_ref[...], preferred_element_type=jnp.float32)
    o_ref[...] = dinv_ref[...] * (acc + b_ref[...])


def _mm_scale(x, w, b, dinv):
    n, k = x.shape
    m = w.shape[1]
    grid = (n // BN,)
    return pl.pallas_call(
        _mm_scale_kernel,
        grid=grid,
        in_specs=[
            pl.BlockSpec((BN, k), lambda i: (i, 0)),
            pl.BlockSpec((k, m), lambda i: (0, 0)),
            pl.BlockSpec((1, m), lambda i: (0, 0)),
            pl.BlockSpec((BN, 1), lambda i: (i, 0)),
        ],
        out_specs=pl.BlockSpec((BN, m), lambda i: (i, 0)),
        out_shape=jax.ShapeDtypeStruct((n, m), jnp.float32),
    )(x, w, b, dinv)


def _relu_mm_scale_kernel(y_ref, w_ref, b_ref, dinv_in_ref, dinv_out_ref, o_ref):
    h = jnp.maximum(dinv_in_ref[...] * y_ref[...], 0.0)
    acc = jnp.dot(h, w_ref[...], preferred_element_type=jnp.float32)
    o_ref[...] = dinv_out_ref[...] * (acc + b_ref[...])


def _relu_mm_scale(y, w, b, dinv_in, dinv_out):
    n, k = y.shape
    m = w.shape[1]
    grid = (n // BN,)
    return pl.pallas_call(
        _relu_mm_scale_kernel,
        grid=grid,
        in_specs=[
            pl.BlockSpec((BN, k), lambda i: (i, 0)),
            pl.BlockSpec((k, m), lambda i: (0, 0)),
            pl.BlockSpec((1, m), lambda i: (0, 0)),
            pl.BlockSpec((BN, 1), lambda i: (i, 0)),
            pl.BlockSpec((BN, 1), lambda i: (i, 0)),
        ],
        out_specs=pl.BlockSpec((BN, m), lambda i: (i, 0)),
        out_shape=jax.ShapeDtypeStruct((n, m), jnp.float32),
    )(y, w, b, dinv_in, dinv_out)


def kernel(x, edge_index, W1, b1, W2, b2, Wd, bd):
    src = edge_index[0]
    dst = edge_index[1]

    deg = jnp.ones((N,), jnp.float32).at[dst].add(1.0)
    dinv = jax.lax.rsqrt(jnp.maximum(deg, 1e-6)).reshape(N, 1)
    ones = jnp.ones((N, 1), jnp.float32)

    z1 = _mm_scale(x, W1, b1.reshape(1, D_H), dinv)
    y1 = z1.at[dst].add(z1[src])
    z2 = _relu_mm_scale(y1, W2, b2.reshape(1, D_H), dinv, dinv)
    y2 = z2.at[dst].add(z2[src])
    logits = _relu_mm_scale(y2, Wd, bd.reshape(1, N_CLS), dinv, ones)
    return logits


# speedup vs baseline: 6.1207x; 2.5693x over previous
import functools

import jax
import jax.numpy as jnp
from jax import lax
from jax.experimental import pallas as pl
from jax.experimental.pallas import tpu as pltpu
from jax.experimental.pallas import tpu_sc as plsc

N = 10000
E = 160000
D_IN = 256
D_H = 512
N_CLS = 40

EP = 163840
W = 640
CW = 128
NCHUNK = D_H // CW
NC2 = NCHUNK // 2
NH = 5120
HALF = 5000
BN = 1000
HIST = 10240

_mesh = plsc.VectorSubcoreMesh(core_axis_name="c", subcore_axis_name="s")


@functools.partial(
    pl.kernel,
    out_type=jax.ShapeDtypeStruct((HIST,), jnp.float32),
    mesh=_mesh,
    scratch_types=[
        pltpu.VMEM((E // 16,), jnp.int32),
        pltpu.VMEM((E // 16,), jnp.float32),
        pltpu.VMEM((HIST // 16,), jnp.float32),
        pltpu.VMEM_SHARED((HIST,), jnp.float32),
        pltpu.SemaphoreType.DMA,
    ],
)
def _deg_kernel(dst_hbm, out_hbm, idx_v, ones_v, z_v, dacc, sem):
    c = lax.axis_index("c")
    s = lax.axis_index("s")
    npt = HIST // 16
    ept = E // 16

    def fill_ones(k, _):
        ones_v[pl.ds(k * 16, 16)] = jnp.ones((16,), jnp.float32)
        return 0

    lax.fori_loop(0, ept // 16, fill_ones, 0)

    def fill_zero(k, _):
        z_v[pl.ds(k * 16, 16)] = jnp.zeros((16,), jnp.float32)
        return 0

    lax.fori_loop(0, npt // 16, fill_zero, 0)
    pltpu.sync_copy(z_v, dacc.at[pl.ds(s * npt, npt)])
    plsc.subcore_barrier()
    pltpu.sync_copy(dst_hbm.at[pl.ds(s * ept, ept)], idx_v)
    pltpu.async_copy(ones_v, dacc.at[idx_v], sem, add=True).wait()
    plsc.subcore_barrier()

    @pl.when(c == 0)
    def _():
        pltpu.sync_copy(dacc.at[pl.ds(s * npt, npt)],
                        out_hbm.at[pl.ds(s * npt, npt)])


@functools.partial(
    pl.kernel,
    out_type=jax.ShapeDtypeStruct((NCHUNK * N, CW), jnp.float32),
    mesh=_mesh,
    scratch_types=[
        pltpu.VMEM((W,), jnp.int32),
        pltpu.VMEM((W,), jnp.int32),
        pltpu.VMEM((W, CW), jnp.float32),
        pltpu.VMEM_SHARED((NH, CW), jnp.float32),
        pltpu.SemaphoreType.DMA,
        pltpu.SemaphoreType.DMA,
    ],
)
def _prop_kernel(z_hbm, src4_hbm, dh0_hbm, dh1_hbm, zrows_hbm, y_hbm,
                 src_v, dst_v, rows_v, acc, gsem, ssem):
    c = lax.axis_index("c")
    s = lax.axis_index("s")
    ept = EP // 16
    nwin = ept // W

    first = True
    for j in range(NC2):
        chunk = c * NC2 + j
        for h in range(2):
            dh_hbm = dh0_hbm if h == 0 else dh1_hbm
            if not first:
                plsc.subcore_barrier()
            first = False

            @pl.when(s < 15)
            def _():
                pltpu.sync_copy(
                    z_hbm.at[pl.ds(chunk * N + h * HALF + s * 312, 312)],
                    acc.at[pl.ds(s * 312, 312)])

            @pl.when(s == 15)
            def _():
                pltpu.sync_copy(
                    z_hbm.at[pl.ds(chunk * N + h * HALF + 4680, 320)],
                    acc.at[pl.ds(4680, 320)])
                pltpu.sync_copy(zrows_hbm, acc.at[pl.ds(HALF, NH - HALF)])

            plsc.subcore_barrier()

            for w in range(nwin):
                eoff = s * ept + w * W
                pltpu.sync_copy(src4_hbm.at[pl.ds(chunk * EP + eoff, W)],
                                src_v)
                pltpu.sync_copy(dh_hbm.at[pl.ds(eoff, W)], dst_v)
                pltpu.async_copy(z_hbm.at[src_v], rows_v, gsem).wait()
                pltpu.async_copy(rows_v, acc.at[dst_v], ssem, add=True).wait()

            plsc.subcore_barrier()

            @pl.when(s < 15)
            def _():
                pltpu.sync_copy(
                    acc.at[pl.ds(s * 312, 312)],
                    y_hbm.at[pl.ds(chunk * N + h * HALF + s * 312, 312)])

            @pl.when(s == 15)
            def _():
                pltpu.sync_copy(
                    acc.at[pl.ds(4680, 320)],
                    y_hbm.at[pl.ds(chunk * N + h * HALF + 4680, 320)])


def _tc1_kernel(x_ref, w_ref, b_ref, deg_ref, z_ref, dinv_ref):
    deg = deg_ref[...] + 1.0
    dinv = lax.rsqrt(jnp.maximum(deg, 1e-6))
    acc = jnp.dot(x_ref[...], w_ref[0], preferred_element_type=jnp.float32)
    z_ref[0] = dinv * (acc + b_ref[0])
    dinv_ref[...] = dinv


def _tc1(x, w, b, deg):
    return pl.pallas_call(
        _tc1_kernel,
        grid=(N // BN, NCHUNK),
        in_specs=[
            pl.BlockSpec((BN, D_IN), lambda i, cc: (i, 0)),
            pl.BlockSpec((1, D_IN, CW), lambda i, cc: (cc, 0, 0)),
            pl.BlockSpec((1, 1, CW), lambda i, cc: (cc, 0, 0)),
            pl.BlockSpec((BN, 1), lambda i, cc: (i, 0)),
        ],
        out_specs=[
            pl.BlockSpec((1, BN, CW), lambda i, cc: (cc, i, 0)),
            pl.BlockSpec((BN, 1), lambda i, cc: (i, 0)),
        ],
        out_shape=[
            jax.ShapeDtypeStruct((NCHUNK, N, CW), jnp.float32),
            jax.ShapeDtypeStruct((N, 1), jnp.float32),
        ],
    )(x, w, b, deg)


def _tc2_kernel(y_ref, w_ref, b_ref, dinv_ref, z_ref):
    k = pl.program_id(2)
    dinv = dinv_ref[...]
    h = jnp.maximum(dinv * y_ref[0], 0.0)
    part = jnp.dot(h, w_ref[0, 0], preferred_element_type=jnp.float32)

    @pl.when(k == 0)
    def _():
        z_ref[0] = part + b_ref[0]

    @pl.when(k > 0)
    def _():
        z_ref[0] += part

    @pl.when(k == NCHUNK - 1)
    def _():
        z_ref[0] *= dinv


def _tc2(y, w, b, dinv):
    return pl.pallas_call(
        _tc2_kernel,
        grid=(N // BN, NCHUNK, NCHUNK),
        in_specs=[
            pl.BlockSpec((1, BN, CW), lambda i, co, k: (k, i, 0)),
            pl.BlockSpec((1, 1, CW, CW), lambda i, co, k: (k, co, 0, 0)),
            pl.BlockSpec((1, 1, CW), lambda i, co, k: (co, 0, 0)),
            pl.BlockSpec((BN, 1), lambda i, co, k: (i, 0)),
        ],
        out_specs=pl.BlockSpec((1, BN, CW), lambda i, co, k: (co, i, 0)),
        out_shape=jax.ShapeDtypeStruct((NCHUNK, N, CW), jnp.float32),
    )(y, w, b, dinv)


def _tc3_kernel(y_ref, w_ref, b_ref, dinv_ref, o_ref):
    k = pl.program_id(1)
    h = jnp.maximum(dinv_ref[...] * y_ref[0], 0.0)
    part = jnp.dot(h, w_ref[0], preferred_element_type=jnp.float32)

    @pl.when(k == 0)
    def _():
        o_ref[...] = part + b_ref[...]

    @pl.when(k > 0)
    def _():
        o_ref[...] += part


def _tc3(y, w, b, dinv):
    return pl.pallas_call(
        _tc3_kernel,
        grid=(N // BN, NCHUNK),
        in_specs=[
            pl.BlockSpec((1, BN, CW), lambda i, k: (k, i, 0)),
            pl.BlockSpec((1, CW, N_CLS), lambda i, k: (k, 0, 0)),
            pl.BlockSpec((1, N_CLS), lambda i, k: (0, 0)),
            pl.BlockSpec((BN, 1), lambda i, k: (i, 0)),
        ],
        out_specs=pl.BlockSpec((BN, N_CLS), lambda i, k: (i, 0)),
        out_shape=jax.ShapeDtypeStruct((N, N_CLS), jnp.float32),
    )(y, w, b, dinv)


def kernel(x, edge_index, W1, b1, W2, b2, Wd, bd):
    src = edge_index[0].astype(jnp.int32)
    dst = edge_index[1].astype(jnp.int32)

    padc = EP - E
    pidx = jnp.arange(padc, dtype=jnp.int32)
    eidx = jnp.arange(EP, dtype=jnp.int32)
    src_p = jnp.concatenate([src, (pidx * 97) % N])
    dst_p = jnp.concatenate([dst, jnp.full((padc,), -1, jnp.int32)])
    dummy = HALF + eidx % (NH - HALF)
    dh0 = jnp.where((dst_p >= 0) & (dst_p < HALF), dst_p, dummy)
    dh1 = jnp.where(dst_p >= HALF, dst_p - HALF, dummy)
    offs = (jnp.arange(NCHUNK, dtype=jnp.int32) * N)[:, None]
    src4 = (src_p[None, :] + offs).reshape(-1)
    zrows = jnp.zeros((NH - HALF, CW), jnp.float32)

    W1r = W1.reshape(D_IN, NCHUNK, CW).transpose(1, 0, 2)
    W2r = W2.reshape(NCHUNK, CW, NCHUNK, CW).transpose(0, 2, 1, 3)

    deg = _deg_kernel(dst)[:N].reshape(N, 1)
    z1, dinv = _tc1(x, W1r, b1.reshape(NCHUNK, 1, CW), deg)
    y1 = _prop_kernel(z1.reshape(NCHUNK * N, CW), src4, dh0, dh1, zrows)
    z2 = _tc2(y1.reshape(NCHUNK, N, CW), W2r, b2.reshape(NCHUNK, 1, CW), dinv)
    y2 = _prop_kernel(z2.reshape(NCHUNK * N, CW), src4, dh0, dh1, zrows)
    logits = _tc3(y2.reshape(NCHUNK, N, CW), Wd.reshape(NCHUNK, CW, N_CLS),
                  bd.reshape(1, N_CLS), dinv)
    return logits


# speedup vs baseline: 7.4464x; 1.2166x over previous
import functools

import jax
import jax.numpy as jnp
from jax import lax
from jax.experimental import pallas as pl
from jax.experimental.pallas import tpu as pltpu
from jax.experimental.pallas import tpu_sc as plsc

N = 10000
E = 160000
D_IN = 256
D_H = 512
N_CLS = 40

EP = 163840
W = 320
CW = 128
NCHUNK = D_H // CW
NC2 = NCHUNK // 2
NH = 5120
HALF = 5000
BN = 1000
HIST = 10240

_mesh = plsc.VectorSubcoreMesh(core_axis_name="c", subcore_axis_name="s")


@functools.partial(
    pl.kernel,
    out_type=jax.ShapeDtypeStruct((HIST,), jnp.float32),
    mesh=_mesh,
    scratch_types=[
        pltpu.VMEM((E // 16,), jnp.int32),
        pltpu.VMEM((E // 16,), jnp.float32),
        pltpu.VMEM((HIST // 16,), jnp.float32),
        pltpu.VMEM_SHARED((HIST,), jnp.float32),
        pltpu.SemaphoreType.DMA,
    ],
)
def _deg_kernel(dst_hbm, out_hbm, idx_v, ones_v, z_v, dacc, sem):
    c = lax.axis_index("c")
    s = lax.axis_index("s")
    npt = HIST // 16
    ept = E // 16

    def fill_ones(k, _):
        ones_v[pl.ds(k * 16, 16)] = jnp.ones((16,), jnp.float32)
        return 0

    lax.fori_loop(0, ept // 16, fill_ones, 0)

    def fill_zero(k, _):
        z_v[pl.ds(k * 16, 16)] = jnp.zeros((16,), jnp.float32)
        return 0

    lax.fori_loop(0, npt // 16, fill_zero, 0)
    pltpu.sync_copy(z_v, dacc.at[pl.ds(s * npt, npt)])
    plsc.subcore_barrier()
    pltpu.sync_copy(dst_hbm.at[pl.ds(s * ept, ept)], idx_v)
    pltpu.async_copy(ones_v, dacc.at[idx_v], sem, add=True).wait()
    plsc.subcore_barrier()

    @pl.when(c == 0)
    def _():
        pltpu.sync_copy(dacc.at[pl.ds(s * npt, npt)],
                        out_hbm.at[pl.ds(s * npt, npt)])


@functools.partial(
    pl.kernel,
    out_type=jax.ShapeDtypeStruct((NCHUNK * N, CW), jnp.float32),
    mesh=_mesh,
    scratch_types=[
        pltpu.VMEM((W,), jnp.int32),
        pltpu.VMEM((W,), jnp.int32),
        pltpu.VMEM((W,), jnp.int32),
        pltpu.VMEM((W,), jnp.int32),
        pltpu.VMEM((W, CW), jnp.float32),
        pltpu.VMEM((W, CW), jnp.float32),
        pltpu.VMEM_SHARED((NH, CW), jnp.float32),
        pltpu.SemaphoreType.DMA,
        pltpu.SemaphoreType.DMA,
        pltpu.SemaphoreType.DMA,
    ],
)
def _prop_kernel(z_hbm, src4_hbm, dh0_hbm, dh1_hbm, zrows_hbm, y_hbm,
                 src_v0, src_v1, dst_v0, dst_v1, rows_v0, rows_v1, acc,
                 isem, gsem, ssem):
    c = lax.axis_index("c")
    s = lax.axis_index("s")
    ept = EP // 16
    nwin = ept // W

    first = True
    for j in range(NC2):
        chunk = c * NC2 + j
        for h in range(2):
            dh_hbm = dh0_hbm if h == 0 else dh1_hbm
            if not first:
                plsc.subcore_barrier()
            first = False

            @pl.when(s < 15)
            def _():
                pltpu.sync_copy(
                    z_hbm.at[pl.ds(chunk * N + h * HALF + s * 312, 312)],
                    acc.at[pl.ds(s * 312, 312)])

            @pl.when(s == 15)
            def _():
                pltpu.sync_copy(
                    z_hbm.at[pl.ds(chunk * N + h * HALF + 4680, 320)],
                    acc.at[pl.ds(4680, 320)])
                pltpu.sync_copy(zrows_hbm, acc.at[pl.ds(HALF, NH - HALF)])

            plsc.subcore_barrier()

            srcb = (src_v0, src_v1)
            dstb = (dst_v0, dst_v1)
            rowsb = (rows_v0, rows_v1)
            ebase = s * ept
            ih = (
                pltpu.async_copy(src4_hbm.at[pl.ds(chunk * EP + ebase, W)],
                                 srcb[0], isem),
                pltpu.async_copy(dh_hbm.at[pl.ds(ebase, W)], dstb[0], isem),
            )
            scat = [None, None]
            for w in range(nwin):
                b = w & 1
                ih[0].wait()
                ih[1].wait()
                if scat[b] is not None:
                    scat[b].wait()
                    scat[b] = None
                pltpu.async_copy(z_hbm.at[srcb[b]], rowsb[b], gsem).wait()
                scat[b] = pltpu.async_copy(rowsb[b], acc.at[dstb[b]], ssem,
                                           add=True)
                if w + 1 < nwin:
                    if scat[1 - b] is not None:
                        scat[1 - b].wait()
                        scat[1 - b] = None
                    eoff = ebase + (w + 1) * W
                    ih = (
                        pltpu.async_copy(
                            src4_hbm.at[pl.ds(chunk * EP + eoff, W)],
                            srcb[1 - b], isem),
                        pltpu.async_copy(dh_hbm.at[pl.ds(eoff, W)],
                                         dstb[1 - b], isem),
                    )
            for sh in scat:
                if sh is not None:
                    sh.wait()

            plsc.subcore_barrier()

            @pl.when(s < 15)
            def _():
                pltpu.sync_copy(
                    acc.at[pl.ds(s * 312, 312)],
                    y_hbm.at[pl.ds(chunk * N + h * HALF + s * 312, 312)])

            @pl.when(s == 15)
            def _():
                pltpu.sync_copy(
                    acc.at[pl.ds(4680, 320)],
                    y_hbm.at[pl.ds(chunk * N + h * HALF + 4680, 320)])


def _tc1_kernel(x_ref, w_ref, b_ref, deg_ref, z_ref, dinv_ref):
    deg = deg_ref[...] + 1.0
    dinv = lax.rsqrt(jnp.maximum(deg, 1e-6))
    acc = jnp.dot(x_ref[...], w_ref[0], preferred_element_type=jnp.float32)
    z_ref[0] = dinv * (acc + b_ref[0])
    dinv_ref[...] = dinv


def _tc1(x, w, b, deg):
    return pl.pallas_call(
        _tc1_kernel,
        grid=(N // BN, NCHUNK),
        in_specs=[
            pl.BlockSpec((BN, D_IN), lambda i, cc: (i, 0)),
            pl.BlockSpec((1, D_IN, CW), lambda i, cc: (cc, 0, 0)),
            pl.BlockSpec((1, 1, CW), lambda i, cc: (cc, 0, 0)),
            pl.BlockSpec((BN, 1), lambda i, cc: (i, 0)),
        ],
        out_specs=[
            pl.BlockSpec((1, BN, CW), lambda i, cc: (cc, i, 0)),
            pl.BlockSpec((BN, 1), lambda i, cc: (i, 0)),
        ],
        out_shape=[
            jax.ShapeDtypeStruct((NCHUNK, N, CW), jnp.float32),
            jax.ShapeDtypeStruct((N, 1), jnp.float32),
        ],
    )(x, w, b, deg)


def _tc2_kernel(y_ref, w_ref, b_ref, dinv_ref, z_ref):
    k = pl.program_id(2)
    dinv = dinv_ref[...]
    h = jnp.maximum(dinv * y_ref[0], 0.0)
    part = jnp.dot(h, w_ref[0, 0], preferred_element_type=jnp.float32)

    @pl.when(k == 0)
    def _():
        z_ref[0] = part + b_ref[0]

    @pl.when(k > 0)
    def _():
        z_ref[0] += part

    @pl.when(k == NCHUNK - 1)
    def _():
        z_ref[0] *= dinv


def _tc2(y, w, b, dinv):
    return pl.pallas_call(
        _tc2_kernel,
        grid=(N // BN, NCHUNK, NCHUNK),
        in_specs=[
            pl.BlockSpec((1, BN, CW), lambda i, co, k: (k, i, 0)),
            pl.BlockSpec((1, 1, CW, CW), lambda i, co, k: (k, co, 0, 0)),
            pl.BlockSpec((1, 1, CW), lambda i, co, k: (co, 0, 0)),
            pl.BlockSpec((BN, 1), lambda i, co, k: (i, 0)),
        ],
        out_specs=pl.BlockSpec((1, BN, CW), lambda i, co, k: (co, i, 0)),
        out_shape=jax.ShapeDtypeStruct((NCHUNK, N, CW), jnp.float32),
    )(y, w, b, dinv)


def _tc3_kernel(y_ref, w_ref, b_ref, dinv_ref, o_ref):
    k = pl.program_id(1)
    h = jnp.maximum(dinv_ref[...] * y_ref[0], 0.0)
    part = jnp.dot(h, w_ref[0], preferred_element_type=jnp.float32)

    @pl.when(k == 0)
    def _():
        o_ref[...] = part + b_ref[...]

    @pl.when(k > 0)
    def _():
        o_ref[...] += part


def _tc3(y, w, b, dinv):
    return pl.pallas_call(
        _tc3_kernel,
        grid=(N // BN, NCHUNK),
        in_specs=[
            pl.BlockSpec((1, BN, CW), lambda i, k: (k, i, 0)),
            pl.BlockSpec((1, CW, N_CLS), lambda i, k: (k, 0, 0)),
            pl.BlockSpec((1, N_CLS), lambda i, k: (0, 0)),
            pl.BlockSpec((BN, 1), lambda i, k: (i, 0)),
        ],
        out_specs=pl.BlockSpec((BN, N_CLS), lambda i, k: (i, 0)),
        out_shape=jax.ShapeDtypeStruct((N, N_CLS), jnp.float32),
    )(y, w, b, dinv)


def kernel(x, edge_index, W1, b1, W2, b2, Wd, bd):
    src = edge_index[0].astype(jnp.int32)
    dst = edge_index[1].astype(jnp.int32)

    padc = EP - E
    pidx = jnp.arange(padc, dtype=jnp.int32)
    eidx = jnp.arange(EP, dtype=jnp.int32)
    src_p = jnp.concatenate([src, (pidx * 97) % N])
    dst_p = jnp.concatenate([dst, jnp.full((padc,), -1, jnp.int32)])
    dummy = HALF + eidx % (NH - HALF)
    dh0 = jnp.where((dst_p >= 0) & (dst_p < HALF), dst_p, dummy)
    dh1 = jnp.where(dst_p >= HALF, dst_p - HALF, dummy)
    offs = (jnp.arange(NCHUNK, dtype=jnp.int32) * N)[:, None]
    src4 = (src_p[None, :] + offs).reshape(-1)
    zrows = jnp.zeros((NH - HALF, CW), jnp.float32)

    W1r = W1.reshape(D_IN, NCHUNK, CW).transpose(1, 0, 2)
    W2r = W2.reshape(NCHUNK, CW, NCHUNK, CW).transpose(0, 2, 1, 3)

    deg = _deg_kernel(dst)[:N].reshape(N, 1)
    z1, dinv = _tc1(x, W1r, b1.reshape(NCHUNK, 1, CW), deg)
    y1 = _prop_kernel(z1.reshape(NCHUNK * N, CW), src4, dh0, dh1, zrows)
    z2 = _tc2(y1.reshape(NCHUNK, N, CW), W2r, b2.reshape(NCHUNK, 1, CW), dinv)
    y2 = _prop_kernel(z2.reshape(NCHUNK * N, CW), src4, dh0, dh1, zrows)
    logits = _tc3(y2.reshape(NCHUNK, N, CW), Wd.reshape(NCHUNK, CW, N_CLS),
                  bd.reshape(1, N_CLS), dinv)
    return logits
